# Initial kernel scaffold; baseline (speedup 1.0000x reference)
import jax
import jax.numpy as jnp
from jax.experimental import pallas as pl


def kernel(x, routing_weights, index, u, svh):
    raise NotImplementedError("write your pallas kernel here")


# speedup vs baseline: 50.1493x; 50.1493x over previous
import functools

import jax
import jax.numpy as jnp
from jax.experimental import pallas as pl

T, D, EK, S = 2048, 1024, 128, 32
TB = 256


def _body(x_ref, idx_ref, u_ref, svh_ref, o_ref):
    x = x_ref[...]
    p = jax.lax.dot_general(x, svh_ref[...], (((1,), (1,)), ((), ())),
                            preferred_element_type=jnp.float32)
    iota = jax.lax.broadcasted_iota(jnp.int32, (TB, EK), 1)
    cnt = jnp.zeros((TB, EK), jnp.float32)
    for s in range(S):
        cnt = cnt + (idx_ref[:, s:s + 1] == iota).astype(jnp.float32)
    scaled = p * cnt
    o_ref[...] = jax.lax.dot_general(scaled, u_ref[...], (((1,), (1,)), ((), ())),
                                     preferred_element_type=jnp.float32)


@jax.jit
def _run(x, index, u, svh):
    grid = (T // TB,)
    return pl.pallas_call(
        _body,
        grid=grid,
        in_specs=[
            pl.BlockSpec((TB, D), lambda i: (i, 0)),
            pl.BlockSpec((TB, S), lambda i: (i, 0)),
            pl.BlockSpec((D, EK), lambda i: (0, 0)),
            pl.BlockSpec((EK, D), lambda i: (0, 0)),
        ],
        out_specs=pl.BlockSpec((TB, D), lambda i: (i, 0)),
        out_shape=jax.ShapeDtypeStruct((T, D), jnp.float32),
    )(x, index, u, svh)


def kernel(x, routing_weights, index, u, svh):
    del routing_weights
    return _run(x, index, u, svh)
